# baseline (speedup 1.0000x reference)
import jax
import jax.numpy as jnp
import numpy as np
from jax.experimental import pallas as pl
from jax.experimental.pallas import tpu as pltpu

B = 1024
D = 512
H = 8
DH = D // H


def _mm_t(a, w):
    return jax.lax.dot_general(a, w, (((1,), (1,)), ((), ())),
                               preferred_element_type=jnp.float32)


def _head_kernel(x_ref, W1_ref, b1_ref, W2_ref, b2_ref,
                 Wq_ref, bq_ref, Wk_ref, Wv_ref, bv_ref,
                 WoT_ref, bo_ref, W3_ref, b3_ref, W4_ref, b4_ref,
                 out_ref, acc_ref):
    i = pl.program_id(0)
    x = x_ref[...]
    scale = np.float32(1.0 / np.sqrt(DH))

    bq = bq_ref[...].reshape(1, DH)
    bv = bv_ref[...].reshape(1, DH)
    qh = (_mm_t(x, Wq_ref[...]) + bq) * scale
    kh = _mm_t(x, Wk_ref[...])
    vh = _mm_t(x, Wv_ref[...]) + bv

    col = jax.lax.broadcasted_iota(jnp.int32, (B, DH), 1)
    ones_blk = (col == 0).astype(jnp.float32)
    vh_aug = jnp.concatenate([vh, ones_blk], axis=-1)

    s = jax.lax.dot_general(qh, kh, (((1,), (1,)), ((), ())),
                            preferred_element_type=jnp.float32)
    m = jnp.max(s, axis=-1, keepdims=True)
    e = jnp.exp(s - m)
    o2 = jnp.dot(e, vh_aug, preferred_element_type=jnp.float32)
    att_h = o2[:, :DH] * (1.0 / o2[:, DH:DH + 1])

    contrib = jax.lax.dot_general(att_h, WoT_ref[...], (((1,), (0,)), ((), ())),
                                  preferred_element_type=jnp.float32)

    @pl.when(i == 0)
    def _init():
        acc_ref[...] = contrib

    @pl.when(i > 0)
    def _accum():
        acc_ref[...] += contrib

    @pl.when(i == H - 1)
    def _finish():
        attended = acc_ref[...] + bo_ref[...]
        h = jnp.maximum(_mm_t(x, W1_ref[...]) + b1_ref[...], 0.0)
        patterns = jnp.maximum(_mm_t(h, W2_ref[...]) + b2_ref[...], 0.0)
        W3 = W3_ref[...]
        h2 = jnp.maximum(_mm_t(patterns, W3[:, :128])
                         + _mm_t(attended, W3[:, 128:]) + b3_ref[...], 0.0)
        out_ref[...] = jnp.tanh(_mm_t(h2, W4_ref[...]) + b4_ref[...])


def kernel(sensory_input, W1, b1, W2, b2, Wq, bq, Wk, bk, Wv, bv, Wo, bo, W3, b3, W4, b4):
    del bk
    full = lambda shape: pl.BlockSpec(shape, lambda i: tuple(0 for _ in shape))
    head_rows = pl.BlockSpec((DH, D), lambda i: (i, 0))
    head_bias = pl.BlockSpec((1, 1, DH), lambda i: (i, 0, 0))
    return pl.pallas_call(
        _head_kernel,
        grid=(H,),
        in_specs=[
            full((B, D)),
            full((256, D)), full((1, 256)),
            full((128, 256)), full((1, 128)),
            head_rows, head_bias,
            head_rows,
            head_rows, head_bias,
            head_rows, full((1, D)),
            full((256, 128 + D)), full((1, 256)),
            full((D, 256)), full((1, D)),
        ],
        out_specs=full((B, D)),
        out_shape=jax.ShapeDtypeStruct((B, D), jnp.float32),
        scratch_shapes=[pltpu.VMEM((B, D), jnp.float32)],
    )(sensory_input, W1, b1.reshape(1, -1), W2, b2.reshape(1, -1),
      Wq, bq.reshape(H, 1, DH), Wk, Wv, bv.reshape(H, 1, DH),
      Wo.T, bo.reshape(1, -1), W3, b3.reshape(1, -1), W4, b4.reshape(1, -1))


# speedup vs baseline: 2.0069x; 2.0069x over previous
import jax
import jax.numpy as jnp
import numpy as np
from jax.experimental import pallas as pl

B = 1024
D = 512
H = 8
DH = D // H


def _mm_t(a, w):
    return jax.lax.dot_general(a, w, (((1,), (1,)), ((), ())),
                               preferred_element_type=jnp.float32)


def _fused_kernel(x_ref, W1_ref, b1_ref, W2_ref, b2_ref,
                  Wq_ref, bq_ref, Wk_ref, Wv_ref, bv_ref,
                  Wo_ref, bo_ref, W3_ref, b3_ref,
                  W4_ref, b4_ref, out_ref):
    x = x_ref[...]
    h = jnp.maximum(_mm_t(x, W1_ref[...]) + b1_ref[...], 0.0)
    patterns = jnp.maximum(_mm_t(h, W2_ref[...]) + b2_ref[...], 0.0)

    scale = np.float32(1.0 / np.sqrt(DH))
    q = (_mm_t(x, Wq_ref[...]) + bq_ref[...]) * scale
    k = _mm_t(x, Wk_ref[...])
    v = _mm_t(x, Wv_ref[...]) + bv_ref[...]

    col = jax.lax.broadcasted_iota(jnp.int32, (B, DH), 1)
    ones_blk = (col == 0).astype(jnp.float32)

    head_outs = []
    for hh in range(H):
        qh = q[:, hh * DH:(hh + 1) * DH]
        kh = k[:, hh * DH:(hh + 1) * DH]
        vh = jnp.concatenate([v[:, hh * DH:(hh + 1) * DH], ones_blk], axis=-1)
        s = jax.lax.dot_general(qh, kh, (((1,), (1,)), ((), ())),
                                preferred_element_type=jnp.float32)
        m = jnp.max(s, axis=-1, keepdims=True)
        e = jnp.exp(s - m)
        o2 = jnp.dot(e, vh, preferred_element_type=jnp.float32)
        r = 1.0 / o2[:, DH:DH + 1]
        head_outs.append(o2[:, :DH] * r)
    att = jnp.concatenate(head_outs, axis=-1)
    attended = _mm_t(att, Wo_ref[...]) + bo_ref[...]

    W3 = W3_ref[...]
    h2 = jnp.maximum(_mm_t(patterns, W3[:, :128])
                     + _mm_t(attended, W3[:, 128:]) + b3_ref[...], 0.0)
    out_ref[...] = jnp.tanh(_mm_t(h2, W4_ref[...]) + b4_ref[...])


def kernel(sensory_input, W1, b1, W2, b2, Wq, bq, Wk, bk, Wv, bv, Wo, bo, W3, b3, W4, b4):
    del bk
    return pl.pallas_call(
        _fused_kernel,
        out_shape=jax.ShapeDtypeStruct((B, D), jnp.float32),
    )(sensory_input, W1, b1, W2, b2, Wq, bq, Wk, Wv, bv, Wo, bo, W3, b3, W4, b4)


# speedup vs baseline: 2.0505x; 1.0217x over previous
import jax
import jax.numpy as jnp
import numpy as np
from jax.experimental import pallas as pl

B = 1024
D = 512
H = 8
DH = D // H


def _mm_t(a, w):
    return jax.lax.dot_general(a, w, (((1,), (1,)), ((), ())),
                               preferred_element_type=jnp.float32)


def _fused_kernel(x_ref, W1_ref, b1_ref, W2_ref, b2_ref,
                  Wq_ref, bq_ref, Wk_ref, Wv_ref, bv_ref,
                  Wo_ref, bo_ref, W3_ref, b3_ref,
                  W4_ref, b4_ref, out_ref):
    x = x_ref[...]
    h = jnp.maximum(_mm_t(x, W1_ref[...]) + b1_ref[...], 0.0)
    patterns = jnp.maximum(_mm_t(h, W2_ref[...]) + b2_ref[...], 0.0)

    scale = np.float32(1.0 / np.sqrt(DH))
    q = (_mm_t(x, Wq_ref[...]) + bq_ref[...]) * scale
    k = _mm_t(x, Wk_ref[...])
    v = _mm_t(x, Wv_ref[...]) + bv_ref[...]

    col = jax.lax.broadcasted_iota(jnp.int32, (B, DH), 1)
    ones_blk = (col == 0).astype(jnp.float32)

    head_outs = []
    for hh in range(H):
        qh = q[:, hh * DH:(hh + 1) * DH]
        kh = k[:, hh * DH:(hh + 1) * DH]
        vh = jnp.concatenate([v[:, hh * DH:(hh + 1) * DH], ones_blk],
                             axis=-1).astype(jnp.bfloat16)
        s = jax.lax.dot_general(qh, kh, (((1,), (1,)), ((), ())),
                                preferred_element_type=jnp.float32).astype(jnp.bfloat16)
        m = jnp.max(s, axis=-1, keepdims=True)
        e = jnp.exp(s - m)
        o2 = jnp.dot(e, vh, preferred_element_type=jnp.float32)
        r = 1.0 / o2[:, DH:DH + 1]
        head_outs.append(o2[:, :DH] * r)
    att = jnp.concatenate(head_outs, axis=-1)
    attended = _mm_t(att, Wo_ref[...]) + bo_ref[...]

    W3 = W3_ref[...]
    h2 = jnp.maximum(_mm_t(patterns, W3[:, :128])
                     + _mm_t(attended, W3[:, 128:]) + b3_ref[...], 0.0)
    out_ref[...] = jnp.tanh(_mm_t(h2, W4_ref[...]) + b4_ref[...])


def kernel(sensory_input, W1, b1, W2, b2, Wq, bq, Wk, bk, Wv, bv, Wo, bo, W3, b3, W4, b4):
    del bk
    return pl.pallas_call(
        _fused_kernel,
        out_shape=jax.ShapeDtypeStruct((B, D), jnp.float32),
    )(sensory_input, W1, b1, W2, b2, Wq, bq, Wk, Wv, bv, Wo, bo, W3, b3, W4, b4)


# speedup vs baseline: 2.0592x; 1.0042x over previous
import jax
import jax.numpy as jnp
import numpy as np
from jax.experimental import pallas as pl

B = 1024
D = 512
H = 8
DH = D // H
BF = jnp.bfloat16


def _mm_t(a, w):
    return jax.lax.dot_general(a, w, (((1,), (1,)), ((), ())),
                               preferred_element_type=jnp.float32)


def _fused_kernel(x_ref, W1_ref, b1_ref, W2_ref, b2_ref,
                  Wq_ref, bq_ref, Wk_ref, Wv_ref, bv_ref,
                  Wo_ref, bo_ref, W3_ref, b3_ref,
                  W4_ref, b4_ref, out_ref):
    x = x_ref[...].astype(BF)
    h = jnp.maximum(_mm_t(x, W1_ref[...].astype(BF)) + b1_ref[...], 0.0).astype(BF)
    patterns = jnp.maximum(_mm_t(h, W2_ref[...].astype(BF)) + b2_ref[...],
                           0.0).astype(BF)

    scale = np.float32(1.0 / np.sqrt(DH))
    q = ((_mm_t(x, Wq_ref[...].astype(BF)) + bq_ref[...]) * scale).astype(BF)
    k = _mm_t(x, Wk_ref[...].astype(BF)).astype(BF)
    v = (_mm_t(x, Wv_ref[...].astype(BF)) + bv_ref[...]).astype(BF)

    col = jax.lax.broadcasted_iota(jnp.int32, (B, DH), 1)
    ones_blk = (col == 0).astype(BF)

    head_outs = []
    for hh in range(H):
        qh = q[:, hh * DH:(hh + 1) * DH]
        kh = k[:, hh * DH:(hh + 1) * DH]
        vh = jnp.concatenate([v[:, hh * DH:(hh + 1) * DH], ones_blk], axis=-1)
        s = jax.lax.dot_general(qh, kh, (((1,), (1,)), ((), ())),
                                preferred_element_type=jnp.float32).astype(BF)
        m = jnp.max(s, axis=-1, keepdims=True)
        e = jnp.exp(s - m)
        o2 = jnp.dot(e, vh, preferred_element_type=jnp.float32)
        r = 1.0 / o2[:, DH:DH + 1]
        head_outs.append((o2[:, :DH] * r).astype(BF))
    att = jnp.concatenate(head_outs, axis=-1)
    attended = (_mm_t(att, Wo_ref[...].astype(BF)) + bo_ref[...]).astype(BF)

    W3 = W3_ref[...].astype(BF)
    h2 = jnp.maximum(_mm_t(patterns, W3[:, :128])
                     + _mm_t(attended, W3[:, 128:]) + b3_ref[...], 0.0)
    out_ref[...] = jnp.tanh(_mm_t(h2.astype(BF), W4_ref[...].astype(BF))
                            + b4_ref[...])


def kernel(sensory_input, W1, b1, W2, b2, Wq, bq, Wk, bk, Wv, bv, Wo, bo, W3, b3, W4, b4):
    del bk
    return pl.pallas_call(
        _fused_kernel,
        out_shape=jax.ShapeDtypeStruct((B, D), jnp.float32),
    )(sensory_input, W1, b1, W2, b2, Wq, bq, Wk, Wv, bv, Wo, bo, W3, b3, W4, b4)
